# initial kernel scaffold (unmeasured)
import jax
import jax.numpy as jnp
from jax import lax
from jax.experimental import pallas as pl
from jax.experimental.pallas import tpu as pltpu

N_DEV = 4
SQ = 256
SKV = 4096
HQ = 8
DH = 128
DM = HQ * DH
SCALE = 0.08838834764831843
NEG = -1e30


def kernel(x, Wq, K_ext, V_ext, Wo):
    xb = x[0].astype(jnp.bfloat16)
    Wqb = Wq.astype(jnp.bfloat16)
    Wob = Wo.astype(jnp.bfloat16)
    Kb = K_ext[0].astype(jnp.bfloat16).transpose(1, 0, 2)
    Vb = V_ext[0].astype(jnp.bfloat16).transpose(1, 0, 2)

    def body(x_ref, wq_ref, k_hbm, v_hbm, wo_ref, out_ref,
             k_comm, v_comm, copy_sems, ksend, krecv, vsend, vrecv):
        my = lax.axis_index("i")
        left = (my - 1) % N_DEV
        right = (my + 1) % N_DEV

        ck = pltpu.make_async_copy(k_hbm, k_comm.at[0], copy_sems.at[0])
        cv = pltpu.make_async_copy(v_hbm, v_comm.at[0], copy_sems.at[1])
        ck.start()
        cv.start()

        barrier = pltpu.get_barrier_semaphore()
        for nbr in (left, right):
            pl.semaphore_signal(
                barrier, inc=1,
                device_id=(nbr,), device_id_type=pl.DeviceIdType.MESH,
            )
        pl.semaphore_wait(barrier, 2)

        q32 = lax.dot_general(
            x_ref[...], wq_ref[...], (((1,), (0,)), ((), ())),
            preferred_element_type=jnp.float32,
        )
        qb16 = q32.astype(jnp.bfloat16)

        qi = lax.broadcasted_iota(jnp.int32, (SQ, SKV), 0) // 64
        kj = (lax.broadcasted_iota(jnp.int32, (SQ, SKV), 1) // 64) % 4
        bias = jnp.where(qi == kj, 0.0, NEG).astype(jnp.float32)

        m = [jnp.full((SQ, 1), NEG, jnp.float32) for _ in range(HQ)]
        l = [jnp.zeros((SQ, 1), jnp.float32) for _ in range(HQ)]
        acc = [jnp.zeros((SQ, DH), jnp.float32) for _ in range(HQ)]

        ck.wait()
        cv.wait()

        for c in range(N_DEV):
            slot = c % 2
            nslot = (c + 1) % 2
            if c < N_DEV - 1:
                rk = pltpu.make_async_remote_copy(
                    src_ref=k_comm.at[slot], dst_ref=k_comm.at[nslot],
                    send_sem=ksend.at[c], recv_sem=krecv.at[c],
                    device_id=(right,), device_id_type=pl.DeviceIdType.MESH,
                )
                rv = pltpu.make_async_remote_copy(
                    src_ref=v_comm.at[slot], dst_ref=v_comm.at[nslot],
                    send_sem=vsend.at[c], recv_sem=vrecv.at[c],
                    device_id=(right,), device_id_type=pl.DeviceIdType.MESH,
                )
                rk.start()
                rv.start()

            for h in range(HQ):
                kh = k_comm[slot, h]
                vh = v_comm[slot, h]
                qh = qb16[:, h * DH:(h + 1) * DH]
                s = lax.dot_general(
                    qh, kh, (((1,), (1,)), ((), ())),
                    preferred_element_type=jnp.float32,
                )
                s = s * SCALE + bias
                mn = jnp.maximum(m[h], jnp.max(s, axis=1, keepdims=True))
                alpha = jnp.exp(m[h] - mn)
                p = jnp.exp(s - mn)
                l[h] = l[h] * alpha + jnp.sum(p, axis=1, keepdims=True)
                acc[h] = acc[h] * alpha + lax.dot_general(
                    p.astype(jnp.bfloat16), vh, (((1,), (0,)), ((), ())),
                    preferred_element_type=jnp.float32,
                )
                m[h] = mn

            if c < N_DEV - 1:
                rk.wait()
                rv.wait()

        ctx = jnp.concatenate(
            [(acc[h] / l[h]).astype(jnp.bfloat16) for h in range(HQ)], axis=1
        )
        out_ref[...] = lax.dot_general(
            ctx, wo_ref[...], (((1,), (0,)), ((), ())),
            preferred_element_type=jnp.float32,
        )

    out = pl.pallas_call(
        body,
        out_shape=jax.ShapeDtypeStruct((SQ, DM), jnp.float32),
        in_specs=[
            pl.BlockSpec(memory_space=pltpu.VMEM),
            pl.BlockSpec(memory_space=pltpu.VMEM),
            pl.BlockSpec(memory_space=pltpu.ANY),
            pl.BlockSpec(memory_space=pltpu.ANY),
            pl.BlockSpec(memory_space=pltpu.VMEM),
        ],
        out_specs=pl.BlockSpec(memory_space=pltpu.VMEM),
        scratch_shapes=[
            pltpu.VMEM((2, HQ, SKV, DH), jnp.bfloat16),
            pltpu.VMEM((2, HQ, SKV, DH), jnp.bfloat16),
            pltpu.SemaphoreType.DMA((2,)),
            pltpu.SemaphoreType.DMA((3,)),
            pltpu.SemaphoreType.DMA((3,)),
            pltpu.SemaphoreType.DMA((3,)),
            pltpu.SemaphoreType.DMA((3,)),
        ],
        compiler_params=pltpu.CompilerParams(collective_id=0),
    )(xb, Wqb, Kb, Vb, Wob)
    return out[None]


# baseline (device time: 604807 ns/iter reference)
import jax
import jax.numpy as jnp
from jax import lax
from jax.experimental import pallas as pl
from jax.experimental.pallas import tpu as pltpu

N_DEV = 4
SQ = 256
SKV = 4096
HQ = 8
DH = 128
DM = HQ * DH
SCALE = 0.08838834764831843
NEG = -1e30


def kernel(x, Wq, K_ext, V_ext, Wo):
    xb = x[0].astype(jnp.bfloat16)
    Wqb = Wq.astype(jnp.bfloat16)
    Wob = Wo.astype(jnp.bfloat16)
    Kb = K_ext[0].astype(jnp.bfloat16).transpose(1, 0, 2)
    Vb = V_ext[0].astype(jnp.bfloat16).transpose(1, 0, 2)

    def body(x_ref, wq_ref, k_hbm, v_hbm, wo_ref, out_ref,
             k_comm, v_comm, copy_sems, ksend, krecv, vsend, vrecv):
        my = lax.axis_index("i")
        left = (my - 1) % N_DEV
        right = (my + 1) % N_DEV

        ck = pltpu.make_async_copy(k_hbm, k_comm.at[0], copy_sems.at[0])
        cv = pltpu.make_async_copy(v_hbm, v_comm.at[0], copy_sems.at[1])
        ck.start()
        cv.start()

        barrier = pltpu.get_barrier_semaphore()
        for nbr in (left, right):
            pl.semaphore_signal(
                barrier, inc=1,
                device_id=(nbr,), device_id_type=pl.DeviceIdType.MESH,
            )
        pl.semaphore_wait(barrier, 2)

        q32 = lax.dot_general(
            x_ref[...], wq_ref[...], (((1,), (0,)), ((), ())),
            preferred_element_type=jnp.float32,
        )
        qb16 = q32.astype(jnp.bfloat16)

        qi = lax.broadcasted_iota(jnp.int32, (SQ, SKV), 0) // 64
        kj = (lax.broadcasted_iota(jnp.int32, (SQ, SKV), 1) // 64) % 4
        bias = jnp.where(qi == kj, 0.0, NEG).astype(jnp.float32)

        m = [jnp.full((SQ, 1), NEG, jnp.float32) for _ in range(HQ)]
        l = [jnp.zeros((SQ, 1), jnp.float32) for _ in range(HQ)]
        acc = [jnp.zeros((SQ, DH), jnp.float32) for _ in range(HQ)]

        ck.wait()
        cv.wait()

        for c in range(N_DEV):
            slot = c % 2
            nslot = (c + 1) % 2
            if c < N_DEV - 1:
                rk = pltpu.make_async_remote_copy(
                    src_ref=k_comm.at[slot], dst_ref=k_comm.at[nslot],
                    send_sem=ksend.at[c], recv_sem=krecv.at[c],
                    device_id=(right,), device_id_type=pl.DeviceIdType.MESH,
                )
                rv = pltpu.make_async_remote_copy(
                    src_ref=v_comm.at[slot], dst_ref=v_comm.at[nslot],
                    send_sem=vsend.at[c], recv_sem=vrecv.at[c],
                    device_id=(right,), device_id_type=pl.DeviceIdType.MESH,
                )
                rk.start()
                rv.start()

            for h in range(HQ):
                kh = k_comm[slot, h]
                vh = v_comm[slot, h]
                qh = qb16[:, h * DH:(h + 1) * DH]
                s = lax.dot_general(
                    qh, kh, (((1,), (1,)), ((), ())),
                    preferred_element_type=jnp.float32,
                )
                s = s * SCALE + bias
                mn = jnp.maximum(m[h], jnp.max(s, axis=1, keepdims=True))
                alpha = jnp.exp(m[h] - mn)
                p = jnp.exp(s - mn)
                l[h] = l[h] * alpha + jnp.sum(p, axis=1, keepdims=True)
                acc[h] = acc[h] * alpha + lax.dot_general(
                    p.astype(jnp.bfloat16), vh, (((1,), (0,)), ((), ())),
                    preferred_element_type=jnp.float32,
                )
                m[h] = mn

            if c < N_DEV - 1:
                rk.wait()
                rv.wait()

        ctx = jnp.concatenate(
            [(acc[h] / l[h]).astype(jnp.bfloat16) for h in range(HQ)], axis=1
        )
        out_ref[...] = lax.dot_general(
            ctx, wo_ref[...], (((1,), (0,)), ((), ())),
            preferred_element_type=jnp.float32,
        )

    out = pl.pallas_call(
        body,
        out_shape=jax.ShapeDtypeStruct((SQ, DM), jnp.float32),
        in_specs=[
            pl.BlockSpec(memory_space=pltpu.VMEM),
            pl.BlockSpec(memory_space=pltpu.VMEM),
            pl.BlockSpec(memory_space=pl.ANY),
            pl.BlockSpec(memory_space=pl.ANY),
            pl.BlockSpec(memory_space=pltpu.VMEM),
        ],
        out_specs=pl.BlockSpec(memory_space=pltpu.VMEM),
        scratch_shapes=[
            pltpu.VMEM((2, HQ, SKV, DH), jnp.bfloat16),
            pltpu.VMEM((2, HQ, SKV, DH), jnp.bfloat16),
            pltpu.SemaphoreType.DMA((2,)),
            pltpu.SemaphoreType.DMA((3,)),
            pltpu.SemaphoreType.DMA((3,)),
            pltpu.SemaphoreType.DMA((3,)),
            pltpu.SemaphoreType.DMA((3,)),
        ],
        compiler_params=pltpu.CompilerParams(
            collective_id=0,
            vmem_limit_bytes=60 * 1024 * 1024,
        ),
    )(xb, Wqb, Kb, Vb, Wob)
    return out[None]


# device time: 106212 ns/iter; 5.6943x vs baseline; 5.6943x over previous
import jax
import jax.numpy as jnp
from jax import lax
from jax.experimental import pallas as pl
from jax.experimental.pallas import tpu as pltpu

N_DEV = 4
SQ = 256
SKV = 4096
NRES = 4
KVR = SKV // NRES
HQ = 8
DH = 128
DM = HQ * DH
PW = DM + 128
SCALE = 0.08838834764831843


def kernel(x, Wq, K_ext, V_ext, Wo):
    xb = x[0].astype(jnp.bfloat16)
    Wqb = Wq.astype(jnp.bfloat16)
    Wob = Wo.astype(jnp.bfloat16)
    Kg = (K_ext[0].astype(jnp.bfloat16)
          .reshape(16, NRES, 64, HQ, DH).transpose(1, 3, 0, 2, 4)
          .reshape(NRES, HQ, KVR, DH))
    Vg = (V_ext[0].astype(jnp.bfloat16)
          .reshape(16, NRES, 64, HQ, DH).transpose(1, 3, 0, 2, 4)
          .reshape(NRES, HQ, KVR, DH))

    def body(x_ref, wq_ref, kg_ref, vg_ref, wo_ref, out_ref,
             q_rel, p_out, p_in, qsend, qrecv, psend, precv):
        my = lax.axis_index("i")
        left = (my - 1) % N_DEV
        right = (my + 1) % N_DEV

        q32 = lax.dot_general(
            x_ref[...], wq_ref[...], (((1,), (0,)), ((), ())),
            preferred_element_type=jnp.float32,
        )
        q_rel[0] = q32.astype(jnp.bfloat16)

        barrier = pltpu.get_barrier_semaphore()
        for _ in range(2):
            for nbr in (left, right):
                pl.semaphore_signal(
                    barrier, inc=1,
                    device_id=(nbr,), device_id_type=pl.DeviceIdType.MESH,
                )
            pl.semaphore_wait(barrier, 2)

        for j in range(1, N_DEV):
            rdma = pltpu.make_async_remote_copy(
                src_ref=q_rel.at[0], dst_ref=q_rel.at[(N_DEV - j) % N_DEV],
                send_sem=qsend.at[j - 1], recv_sem=qrecv.at[j - 1],
                device_id=((my + j) % N_DEV,),
                device_id_type=pl.DeviceIdType.MESH,
            )
            rdma.start()
            rdma.wait()

        for r in range(NRES):
            accs, ms, ls = [], [], []
            for h in range(HQ):
                qrh = q_rel[:, r * 64:(r + 1) * 64, h * DH:(h + 1) * DH]
                qrh = qrh.reshape(N_DEV * 64, DH)
                s = lax.dot_general(
                    qrh, kg_ref[r, h], (((1,), (1,)), ((), ())),
                    preferred_element_type=jnp.float32,
                ) * SCALE
                m = jnp.max(s, axis=1, keepdims=True)
                p = jnp.exp(s - m)
                l = jnp.sum(p, axis=1, keepdims=True)
                a = lax.dot_general(
                    p.astype(jnp.bfloat16), vg_ref[r, h],
                    (((1,), (0,)), ((), ())),
                    preferred_element_type=jnp.float32,
                )
                accs.append(a.astype(jnp.bfloat16))
                ms.append(m)
                ls.append(l)
            stat_r = jnp.concatenate(ms + ls, axis=1)
            row_r = jnp.concatenate(
                accs
                + [stat_r.astype(jnp.bfloat16),
                   jnp.zeros((N_DEV * 64, 128 - 2 * HQ), jnp.bfloat16)],
                axis=1,
            )
            p_out[:, r] = row_r.reshape(N_DEV, 64, PW)

        p_in[0] = p_out[0]
        for j in range(1, N_DEV):
            rdma = pltpu.make_async_remote_copy(
                src_ref=p_out.at[j], dst_ref=p_in.at[(N_DEV - j) % N_DEV],
                send_sem=psend.at[j - 1], recv_sem=precv.at[j - 1],
                device_id=((my + j) % N_DEV,),
                device_id_type=pl.DeviceIdType.MESH,
            )
            rdma.start()
            rdma.wait()

        ctx_rows = []
        for r in range(NRES):
            stats = [p_in[u, r, :, DM:DM + 2 * HQ].astype(jnp.float32)
                     for u in range(N_DEV)]
            row_h = []
            for h in range(HQ):
                m_s = [st[:, h:h + 1] for st in stats]
                l_s = [st[:, HQ + h:HQ + h + 1] for st in stats]
                mm = jnp.maximum(jnp.maximum(m_s[0], m_s[1]),
                                 jnp.maximum(m_s[2], m_s[3]))
                w = [jnp.exp(m_s[u] - mm) for u in range(N_DEV)]
                l_tot = sum(w[u] * l_s[u] for u in range(N_DEV))
                a_tot = sum(
                    w[u] * p_in[u, r, :, h * DH:(h + 1) * DH]
                    .astype(jnp.float32)
                    for u in range(N_DEV)
                )
                row_h.append((a_tot / l_tot).astype(jnp.bfloat16))
            ctx_rows.append(jnp.concatenate(row_h, axis=1))
        ctx = jnp.concatenate(ctx_rows, axis=0)

        out_ref[...] = lax.dot_general(
            ctx, wo_ref[...], (((1,), (0,)), ((), ())),
            preferred_element_type=jnp.float32,
        )

    out = pl.pallas_call(
        body,
        out_shape=jax.ShapeDtypeStruct((SQ, DM), jnp.float32),
        in_specs=[pl.BlockSpec(memory_space=pltpu.VMEM)] * 5,
        out_specs=pl.BlockSpec(memory_space=pltpu.VMEM),
        scratch_shapes=[
            pltpu.VMEM((N_DEV, SQ, DM), jnp.bfloat16),
            pltpu.VMEM((N_DEV, NRES, 64, PW), jnp.bfloat16),
            pltpu.VMEM((N_DEV, NRES, 64, PW), jnp.bfloat16),
            pltpu.SemaphoreType.DMA((N_DEV - 1,)),
            pltpu.SemaphoreType.DMA((N_DEV - 1,)),
            pltpu.SemaphoreType.DMA((N_DEV - 1,)),
            pltpu.SemaphoreType.DMA((N_DEV - 1,)),
        ],
        compiler_params=pltpu.CompilerParams(
            collective_id=0,
            vmem_limit_bytes=60 * 1024 * 1024,
        ),
    )(xb, Wqb, Kg, Vg, Wob)
    return out[None]


# device time: 87173 ns/iter; 6.9380x vs baseline; 1.2184x over previous
import jax
import jax.numpy as jnp
from jax import lax
from jax.experimental import pallas as pl
from jax.experimental.pallas import tpu as pltpu

N_DEV = 4
SQ = 256
SKV = 4096
NRES = 4
KVR = SKV // NRES
HQ = 8
DH = 128
DM = HQ * DH
PW = DM + 128
SCALE = 0.08838834764831843


def kernel(x, Wq, K_ext, V_ext, Wo):
    xb = x[0].astype(jnp.bfloat16)
    Wqb = Wq.astype(jnp.bfloat16)
    Wob = Wo.astype(jnp.bfloat16)
    Kg = (K_ext[0].astype(jnp.bfloat16)
          .reshape(16, NRES, 64, HQ, DH).transpose(1, 3, 0, 2, 4)
          .reshape(NRES, HQ, KVR, DH))
    Vg = (V_ext[0].astype(jnp.bfloat16)
          .reshape(16, NRES, 64, HQ, DH).transpose(1, 3, 0, 2, 4)
          .reshape(NRES, HQ, KVR, DH))

    def body(x_ref, wq_ref, kg_ref, vg_ref, wo_ref, out_ref,
             q_rel, p_out, p_in, qsend, qrecv, psend, precv):
        my = lax.axis_index("i")
        left = (my - 1) % N_DEV
        right = (my + 1) % N_DEV

        q32 = lax.dot_general(
            x_ref[...], wq_ref[...], (((1,), (0,)), ((), ())),
            preferred_element_type=jnp.float32,
        )
        q_rel[0] = q32.astype(jnp.bfloat16)

        barrier = pltpu.get_barrier_semaphore()
        for _ in range(2):
            for nbr in (left, right):
                pl.semaphore_signal(
                    barrier, inc=1,
                    device_id=(nbr,), device_id_type=pl.DeviceIdType.MESH,
                )
            pl.semaphore_wait(barrier, 2)

        q_rdmas = []
        for j in range(1, N_DEV):
            rdma = pltpu.make_async_remote_copy(
                src_ref=q_rel.at[0], dst_ref=q_rel.at[(N_DEV - j) % N_DEV],
                send_sem=qsend.at[j - 1], recv_sem=qrecv.at[j - 1],
                device_id=((my + j) % N_DEV,),
                device_id_type=pl.DeviceIdType.MESH,
            )
            rdma.start()
            q_rdmas.append(rdma)
        for rdma in q_rdmas:
            rdma.wait()

        for r in range(NRES):
            accs, ms, ls = [], [], []
            for h in range(HQ):
                qrh = q_rel[:, r * 64:(r + 1) * 64, h * DH:(h + 1) * DH]
                qrh = qrh.reshape(N_DEV * 64, DH)
                s = lax.dot_general(
                    qrh, kg_ref[r, h], (((1,), (1,)), ((), ())),
                    preferred_element_type=jnp.float32,
                ) * SCALE
                m = jnp.max(s, axis=1, keepdims=True)
                p = jnp.exp(s - m)
                l = jnp.sum(p, axis=1, keepdims=True)
                a = lax.dot_general(
                    p.astype(jnp.bfloat16), vg_ref[r, h],
                    (((1,), (0,)), ((), ())),
                    preferred_element_type=jnp.float32,
                )
                accs.append(a.astype(jnp.bfloat16))
                ms.append(m)
                ls.append(l)
            stat_r = jnp.concatenate(ms + ls, axis=1)
            row_r = jnp.concatenate(
                accs
                + [stat_r.astype(jnp.bfloat16),
                   jnp.zeros((N_DEV * 64, 128 - 2 * HQ), jnp.bfloat16)],
                axis=1,
            )
            p_out[:, r] = row_r.reshape(N_DEV, 64, PW)

        p_rdmas = []
        for j in range(1, N_DEV):
            rdma = pltpu.make_async_remote_copy(
                src_ref=p_out.at[j], dst_ref=p_in.at[(N_DEV - j) % N_DEV],
                send_sem=psend.at[j - 1], recv_sem=precv.at[j - 1],
                device_id=((my + j) % N_DEV,),
                device_id_type=pl.DeviceIdType.MESH,
            )
            rdma.start()
            p_rdmas.append(rdma)
        p_in[0] = p_out[0]
        for rdma in p_rdmas:
            rdma.wait()

        ctx_rows = []
        for r in range(NRES):
            stats = [p_in[u, r, :, DM:DM + 2 * HQ].astype(jnp.float32)
                     for u in range(N_DEV)]
            row_h = []
            for h in range(HQ):
                m_s = [st[:, h:h + 1] for st in stats]
                l_s = [st[:, HQ + h:HQ + h + 1] for st in stats]
                mm = jnp.maximum(jnp.maximum(m_s[0], m_s[1]),
                                 jnp.maximum(m_s[2], m_s[3]))
                w = [jnp.exp(m_s[u] - mm) for u in range(N_DEV)]
                l_tot = sum(w[u] * l_s[u] for u in range(N_DEV))
                a_tot = sum(
                    w[u] * p_in[u, r, :, h * DH:(h + 1) * DH]
                    .astype(jnp.float32)
                    for u in range(N_DEV)
                )
                row_h.append((a_tot / l_tot).astype(jnp.bfloat16))
            ctx_rows.append(jnp.concatenate(row_h, axis=1))
        ctx = jnp.concatenate(ctx_rows, axis=0)

        out_ref[...] = lax.dot_general(
            ctx, wo_ref[...], (((1,), (0,)), ((), ())),
            preferred_element_type=jnp.float32,
        )

    out = pl.pallas_call(
        body,
        out_shape=jax.ShapeDtypeStruct((SQ, DM), jnp.float32),
        in_specs=[pl.BlockSpec(memory_space=pltpu.VMEM)] * 5,
        out_specs=pl.BlockSpec(memory_space=pltpu.VMEM),
        scratch_shapes=[
            pltpu.VMEM((N_DEV, SQ, DM), jnp.bfloat16),
            pltpu.VMEM((N_DEV, NRES, 64, PW), jnp.bfloat16),
            pltpu.VMEM((N_DEV, NRES, 64, PW), jnp.bfloat16),
            pltpu.SemaphoreType.DMA((N_DEV - 1,)),
            pltpu.SemaphoreType.DMA((N_DEV - 1,)),
            pltpu.SemaphoreType.DMA((N_DEV - 1,)),
            pltpu.SemaphoreType.DMA((N_DEV - 1,)),
        ],
        compiler_params=pltpu.CompilerParams(
            collective_id=0,
            vmem_limit_bytes=60 * 1024 * 1024,
        ),
    )(xb, Wqb, Kg, Vg, Wob)
    return out[None]


# device time: 87006 ns/iter; 6.9513x vs baseline; 1.0019x over previous
import jax
import jax.numpy as jnp
from jax import lax
from jax.experimental import pallas as pl
from jax.experimental.pallas import tpu as pltpu

N_DEV = 4
SQ = 256
SKV = 4096
NRES = 4
KVR = SKV // NRES
HQ = 8
DH = 128
DM = HQ * DH
PW = DM + 128
SCALE = 0.08838834764831843


def kernel(x, Wq, K_ext, V_ext, Wo):
    xb = x[0].astype(jnp.bfloat16)
    Wqb = Wq.astype(jnp.bfloat16)
    Wob = Wo.astype(jnp.bfloat16)
    Kg = (K_ext[0].astype(jnp.bfloat16)
          .reshape(16, NRES, 64, HQ, DH).transpose(1, 3, 0, 2, 4)
          .reshape(NRES, HQ, KVR, DH))
    Vg = (V_ext[0].astype(jnp.bfloat16)
          .reshape(16, NRES, 64, HQ, DH).transpose(1, 3, 0, 2, 4)
          .reshape(NRES, HQ, KVR, DH))

    def body(x_ref, wq_ref, kg_ref, vg_ref, wo_ref, out_ref,
             q_rel, p_out, p_in, qsend, qrecv, psend, precv):
        my = lax.axis_index("i")
        left = (my - 1) % N_DEV
        right = (my + 1) % N_DEV

        q32 = lax.dot_general(
            x_ref[...], wq_ref[...], (((1,), (0,)), ((), ())),
            preferred_element_type=jnp.float32,
        )
        q_rel[0] = (q32 * SCALE).astype(jnp.bfloat16)

        barrier = pltpu.get_barrier_semaphore()
        for _ in range(2):
            for nbr in (left, right):
                pl.semaphore_signal(
                    barrier, inc=1,
                    device_id=(nbr,), device_id_type=pl.DeviceIdType.MESH,
                )
            pl.semaphore_wait(barrier, 2)

        q_rdmas = []
        for j in range(1, N_DEV):
            rdma = pltpu.make_async_remote_copy(
                src_ref=q_rel.at[0], dst_ref=q_rel.at[(N_DEV - j) % N_DEV],
                send_sem=qsend.at[j - 1], recv_sem=qrecv.at[j - 1],
                device_id=((my + j) % N_DEV,),
                device_id_type=pl.DeviceIdType.MESH,
            )
            rdma.start()
            q_rdmas.append(rdma)
        for rdma in q_rdmas:
            rdma.wait()

        for r in range(NRES):
            accs, ms, ls = [], [], []
            for h in range(HQ):
                qrh = q_rel[:, r * 64:(r + 1) * 64, h * DH:(h + 1) * DH]
                qrh = qrh.reshape(N_DEV * 64, DH)
                s = lax.dot_general(
                    qrh, kg_ref[r, h], (((1,), (1,)), ((), ())),
                    preferred_element_type=jnp.float32,
                )
                m = jnp.max(s, axis=1, keepdims=True)
                p = jnp.exp(s - m)
                l = jnp.sum(p, axis=1, keepdims=True)
                a = lax.dot_general(
                    p.astype(jnp.bfloat16), vg_ref[r, h],
                    (((1,), (0,)), ((), ())),
                    preferred_element_type=jnp.float32,
                )
                accs.append(a.astype(jnp.bfloat16))
                ms.append(m)
                ls.append(l)
            stat_r = jnp.concatenate(ms + ls, axis=1)
            row_r = jnp.concatenate(
                accs
                + [stat_r.astype(jnp.bfloat16),
                   jnp.zeros((N_DEV * 64, 128 - 2 * HQ), jnp.bfloat16)],
                axis=1,
            )
            p_out[:, r] = row_r.reshape(N_DEV, 64, PW)

        p_rdmas = []
        for j in range(1, N_DEV):
            rdma = pltpu.make_async_remote_copy(
                src_ref=p_out.at[j], dst_ref=p_in.at[(N_DEV - j) % N_DEV],
                send_sem=psend.at[j - 1], recv_sem=precv.at[j - 1],
                device_id=((my + j) % N_DEV,),
                device_id_type=pl.DeviceIdType.MESH,
            )
            rdma.start()
            p_rdmas.append(rdma)
        p_in[0] = p_out[0]
        for rdma in p_rdmas:
            rdma.wait()

        ctx_rows = []
        for r in range(NRES):
            stats = [p_in[u, r, :, DM:DM + 2 * HQ].astype(jnp.float32)
                     for u in range(N_DEV)]
            row_h = []
            for h in range(HQ):
                m_s = [st[:, h:h + 1] for st in stats]
                l_s = [st[:, HQ + h:HQ + h + 1] for st in stats]
                mm = jnp.maximum(jnp.maximum(m_s[0], m_s[1]),
                                 jnp.maximum(m_s[2], m_s[3]))
                w = [jnp.exp(m_s[u] - mm) for u in range(N_DEV)]
                l_tot = sum(w[u] * l_s[u] for u in range(N_DEV))
                a_tot = sum(
                    w[u] * p_in[u, r, :, h * DH:(h + 1) * DH]
                    .astype(jnp.float32)
                    for u in range(N_DEV)
                )
                row_h.append((a_tot / l_tot).astype(jnp.bfloat16))
            ctx_rows.append(jnp.concatenate(row_h, axis=1))
        ctx = jnp.concatenate(ctx_rows, axis=0)

        out_ref[...] = lax.dot_general(
            ctx, wo_ref[...], (((1,), (0,)), ((), ())),
            preferred_element_type=jnp.float32,
        )

    out = pl.pallas_call(
        body,
        out_shape=jax.ShapeDtypeStruct((SQ, DM), jnp.float32),
        in_specs=[pl.BlockSpec(memory_space=pltpu.VMEM)] * 5,
        out_specs=pl.BlockSpec(memory_space=pltpu.VMEM),
        scratch_shapes=[
            pltpu.VMEM((N_DEV, SQ, DM), jnp.bfloat16),
            pltpu.VMEM((N_DEV, NRES, 64, PW), jnp.bfloat16),
            pltpu.VMEM((N_DEV, NRES, 64, PW), jnp.bfloat16),
            pltpu.SemaphoreType.DMA((N_DEV - 1,)),
            pltpu.SemaphoreType.DMA((N_DEV - 1,)),
            pltpu.SemaphoreType.DMA((N_DEV - 1,)),
            pltpu.SemaphoreType.DMA((N_DEV - 1,)),
        ],
        compiler_params=pltpu.CompilerParams(
            collective_id=0,
            vmem_limit_bytes=60 * 1024 * 1024,
        ),
    )(xb, Wqb, Kg, Vg, Wob)
    return out[None]


# device time: 78253 ns/iter; 7.7289x vs baseline; 1.1119x over previous
import jax
import jax.numpy as jnp
from jax import lax
from jax.experimental import pallas as pl
from jax.experimental.pallas import tpu as pltpu

N_DEV = 4
SQ = 256
SKV = 4096
NRES = 4
KVR = SKV // NRES
HQ = 8
DH = 128
DM = HQ * DH
PW = DM + 128
SCALE = 0.08838834764831843


def kernel(x, Wq, K_ext, V_ext, Wo):
    xb = x[0].astype(jnp.bfloat16)
    Wqb = Wq.astype(jnp.bfloat16)
    Wob = Wo.astype(jnp.bfloat16)
    Kg = (K_ext[0].astype(jnp.bfloat16)
          .reshape(16, NRES, 64, HQ, DH).transpose(1, 3, 0, 2, 4)
          .reshape(NRES, HQ, KVR, DH))
    Vg = (V_ext[0].astype(jnp.bfloat16)
          .reshape(16, NRES, 64, HQ, DH).transpose(1, 3, 0, 2, 4)
          .reshape(NRES, HQ, KVR, DH))

    def body(x_ref, wq_ref, kg_ref, vg_ref, wo_ref, out_ref,
             q_rel, p_out, p_in, qsend, qrecv, psend, precv):
        my = lax.axis_index("i")
        left = (my - 1) % N_DEV
        right = (my + 1) % N_DEV

        q32 = lax.dot_general(
            x_ref[...], wq_ref[...], (((1,), (0,)), ((), ())),
            preferred_element_type=jnp.float32,
        )
        q_rel[0] = (q32 * SCALE).astype(jnp.bfloat16)

        barrier = pltpu.get_barrier_semaphore()
        for _ in range(2):
            for nbr in (left, right):
                pl.semaphore_signal(
                    barrier, inc=1,
                    device_id=(nbr,), device_id_type=pl.DeviceIdType.MESH,
                )
            pl.semaphore_wait(barrier, 2)

        q_rdmas = []
        for j in range(1, N_DEV):
            rdma = pltpu.make_async_remote_copy(
                src_ref=q_rel.at[0], dst_ref=q_rel.at[(N_DEV - j) % N_DEV],
                send_sem=qsend.at[j - 1], recv_sem=qrecv.at[j - 1],
                device_id=((my + j) % N_DEV,),
                device_id_type=pl.DeviceIdType.MESH,
            )
            rdma.start()
            q_rdmas.append(rdma)
        for rdma in q_rdmas:
            rdma.wait()

        p_rdmas = []
        for r in range(NRES):
            accs, ms, ls = [], [], []
            for h in range(HQ):
                qrh = q_rel[:, r * 64:(r + 1) * 64, h * DH:(h + 1) * DH]
                qrh = qrh.reshape(N_DEV * 64, DH)
                s = lax.dot_general(
                    qrh, kg_ref[r, h], (((1,), (1,)), ((), ())),
                    preferred_element_type=jnp.float32,
                )
                m = jnp.max(s, axis=1, keepdims=True)
                p = jnp.exp(s - m)
                l = jnp.sum(p, axis=1, keepdims=True)
                a = lax.dot_general(
                    p.astype(jnp.bfloat16), vg_ref[r, h],
                    (((1,), (0,)), ((), ())),
                    preferred_element_type=jnp.float32,
                )
                accs.append(a.astype(jnp.bfloat16))
                ms.append(m)
                ls.append(l)
            stat_r = jnp.concatenate(ms + ls, axis=1)
            row_r = jnp.concatenate(
                accs
                + [stat_r.astype(jnp.bfloat16),
                   jnp.zeros((N_DEV * 64, 128 - 2 * HQ), jnp.bfloat16)],
                axis=1,
            )
            p_out[:, r] = row_r.reshape(N_DEV, 64, PW)

            for j in range(1, N_DEV):
                rdma = pltpu.make_async_remote_copy(
                    src_ref=p_out.at[j, r],
                    dst_ref=p_in.at[(N_DEV - j) % N_DEV, r],
                    send_sem=psend.at[j - 1, r], recv_sem=precv.at[j - 1, r],
                    device_id=((my + j) % N_DEV,),
                    device_id_type=pl.DeviceIdType.MESH,
                )
                rdma.start()
                p_rdmas.append(rdma)
            p_in[0, r] = p_out[0, r]

        for rdma in p_rdmas:
            rdma.wait()

        ctx_rows = []
        for r in range(NRES):
            stats = [p_in[u, r, :, DM:DM + 2 * HQ].astype(jnp.float32)
                     for u in range(N_DEV)]
            row_h = []
            for h in range(HQ):
                m_s = [st[:, h:h + 1] for st in stats]
                l_s = [st[:, HQ + h:HQ + h + 1] for st in stats]
                mm = jnp.maximum(jnp.maximum(m_s[0], m_s[1]),
                                 jnp.maximum(m_s[2], m_s[3]))
                w = [jnp.exp(m_s[u] - mm) for u in range(N_DEV)]
                l_tot = sum(w[u] * l_s[u] for u in range(N_DEV))
                a_tot = sum(
                    w[u] * p_in[u, r, :, h * DH:(h + 1) * DH]
                    .astype(jnp.float32)
                    for u in range(N_DEV)
                )
                row_h.append((a_tot / l_tot).astype(jnp.bfloat16))
            ctx_rows.append(jnp.concatenate(row_h, axis=1))
        ctx = jnp.concatenate(ctx_rows, axis=0)

        out_ref[...] = lax.dot_general(
            ctx, wo_ref[...], (((1,), (0,)), ((), ())),
            preferred_element_type=jnp.float32,
        )

    out = pl.pallas_call(
        body,
        out_shape=jax.ShapeDtypeStruct((SQ, DM), jnp.float32),
        in_specs=[pl.BlockSpec(memory_space=pltpu.VMEM)] * 5,
        out_specs=pl.BlockSpec(memory_space=pltpu.VMEM),
        scratch_shapes=[
            pltpu.VMEM((N_DEV, SQ, DM), jnp.bfloat16),
            pltpu.VMEM((N_DEV, NRES, 64, PW), jnp.bfloat16),
            pltpu.VMEM((N_DEV, NRES, 64, PW), jnp.bfloat16),
            pltpu.SemaphoreType.DMA((N_DEV - 1,)),
            pltpu.SemaphoreType.DMA((N_DEV - 1,)),
            pltpu.SemaphoreType.DMA((N_DEV - 1, NRES)),
            pltpu.SemaphoreType.DMA((N_DEV - 1, NRES)),
        ],
        compiler_params=pltpu.CompilerParams(
            collective_id=0,
            vmem_limit_bytes=60 * 1024 * 1024,
        ),
    )(xb, Wqb, Kg, Vg, Wob)
    return out[None]
